# baseline (speedup 1.0000x reference)
import functools

import jax
import jax.numpy as jnp
from jax import lax
from jax.experimental import pallas as pl
from jax.experimental.pallas import tpu as pltpu
from jax.experimental.pallas import tpu_sc as plsc

_ROWS, _COLS = 4096, 16384
_NC, _NS = 2, 16
_NW = _NC * _NS
_LANES = 16
_BLK = 4 * _LANES
_NSLOT = 4
_CH = _COLS // 2

_TC_ROWS = 2048
_SC_ROWS = _ROWS - _TC_ROWS
_BR = 64


def _mask_row(in_ref, out_ref):
    ia = lax.iota(jnp.int32, _LANES) * 4
    ib = ia + 1
    ic = ia + 2
    id_ = ia + 3
    one = jnp.float32(1.0)
    zero = jnp.float32(0.0)

    def block(blk, carry):
        base = blk * _BLK
        in_blk = in_ref.at[pl.ds(base, _BLK)]
        out_blk = out_ref.at[pl.ds(base, _BLK)]
        a = plsc.load_gather(in_blk, [ia])
        b = plsc.load_gather(in_blk, [ib])
        c = plsc.load_gather(in_blk, [ic])
        d = plsc.load_gather(in_blk, [id_])
        x1 = a >= b
        x6 = c >= d
        h1 = jnp.where(x1, a, b)
        l1 = jnp.where(x1, b, a)
        h2 = jnp.where(x6, c, d)
        l2 = jnp.where(x6, d, c)
        y = h1 >= h2
        u = l1 >= h2
        v = h1 >= l2
        kh1 = y | v
        kl1 = y & u
        h1o = jnp.where(kh1, one, zero)
        l1o = jnp.where(kl1, one, zero)
        h2o = jnp.where(kl1, zero, one)
        l2o = jnp.where(kh1, zero, one)
        plsc.store_scatter(out_blk, [ia], jnp.where(x1, h1o, l1o))
        plsc.store_scatter(out_blk, [ib], jnp.where(x1, l1o, h1o))
        plsc.store_scatter(out_blk, [ic], jnp.where(x6, h2o, l2o))
        plsc.store_scatter(out_blk, [id_], jnp.where(x6, l2o, h2o))
        return carry

    lax.fori_loop(0, in_ref.shape[0] // _BLK, block, 0)


def _make_sc(row_offset, sc_rows):
    rpw = sc_rows // _NW
    nj = rpw * 2 // _NSLOT

    @functools.partial(
        pl.kernel,
        out_type=jax.ShapeDtypeStruct((sc_rows, _COLS), jnp.float32),
        mesh=plsc.VectorSubcoreMesh(core_axis_name="c", subcore_axis_name="s"),
        compiler_params=pltpu.CompilerParams(needs_layout_passes=False),
        scratch_types=(
            [pltpu.VMEM((_CH,), jnp.float32)] * (2 * _NSLOT)
            + [pltpu.SemaphoreType.DMA] * (2 * _NSLOT)
        ),
    )
    def _sc(in_hbm, out_hbm, *bufs_and_sems):
        ibufs = bufs_and_sems[:_NSLOT]
        obufs = bufs_and_sems[_NSLOT:2 * _NSLOT]
        isems = bufs_and_sems[2 * _NSLOT:3 * _NSLOT]
        osems = bufs_and_sems[3 * _NSLOT:]
        wid = lax.axis_index("s") * _NC + lax.axis_index("c")
        row0 = wid * rpw

        def cref(hbm, off, j, s):
            return hbm.at[
                off + row0 + 2 * j + s // 2, pl.ds((s % 2) * _CH, _CH)
            ]

        for s in range(_NSLOT):
            pltpu.async_copy(cref(in_hbm, row_offset, 0, s), ibufs[s], isems[s])

        def step(j, carry):
            for s in range(_NSLOT):
                pltpu.make_async_copy(
                    cref(in_hbm, row_offset, j, s), ibufs[s], isems[s]
                ).wait()

                @pl.when(j > 0)
                def _wait_prev_out():
                    pltpu.make_async_copy(
                        obufs[s], cref(out_hbm, 0, j - 1, s), osems[s]
                    ).wait()

                _mask_row(ibufs[s], obufs[s])
                pltpu.async_copy(obufs[s], cref(out_hbm, 0, j, s), osems[s])

                @pl.when(j + 1 < nj)
                def _prefetch_next_in():
                    pltpu.async_copy(
                        cref(in_hbm, row_offset, j + 1, s), ibufs[s], isems[s]
                    )

            return carry

        lax.fori_loop(0, nj, step, 0)
        for s in range(_NSLOT):
            pltpu.make_async_copy(
                obufs[s], cref(out_hbm, 0, nj - 1, s), osems[s]
            ).wait()

    return _sc


def _tc_block(x_ref, o_ref):
    x = x_ref[...]
    p = lax.broadcasted_iota(jnp.int32, x.shape, 1) % 4
    r = lambda k: jnp.roll(x, k, axis=1)
    bm1 = r(1) >= x
    bm2 = r(2) >= x
    bm3 = r(3) >= x
    bp1 = r(-1) > x
    bp2 = r(-2) > x
    bp3 = r(-3) > x
    one = jnp.float32(1.0)
    zero = jnp.float32(0.0)
    c1 = jnp.where((p >= 1) & bm1, one, zero) + jnp.where(
        (p <= 2) & bp1, one, zero
    )
    c2 = jnp.where(((p >= 2) & bm2) | ((p < 2) & bp2), one, zero)
    c3 = jnp.where((p == 3) & bm3, one, zero) + jnp.where(
        (p == 0) & bp3, one, zero
    )
    count = c1 + c2 + c3
    o_ref[...] = jnp.where(count < 1.5, one, zero)


def _make_tc(tc_rows):
    return pl.pallas_call(
        _tc_block,
        grid=(tc_rows // _BR,),
        in_specs=[pl.BlockSpec((_BR, _COLS), lambda i: (i, 0))],
        out_specs=pl.BlockSpec((_BR, _COLS), lambda i: (i, 0)),
        out_shape=jax.ShapeDtypeStruct((tc_rows, _COLS), jnp.float32),
    )


def kernel(mask_scores, ampere_temperature):
    del ampere_temperature
    tc_out = _make_tc(_TC_ROWS)(mask_scores)
    sc_out = _make_sc(_TC_ROWS, _SC_ROWS)(mask_scores)
    return lax.concatenate([tc_out, sc_out], 0)


# speedup vs baseline: 2.4853x; 2.4853x over previous
import functools

import jax
import jax.numpy as jnp
from jax import lax
from jax.experimental import pallas as pl
from jax.experimental.pallas import tpu as pltpu
from jax.experimental.pallas import tpu_sc as plsc

_ROWS, _COLS = 4096, 16384
_NC, _NS = 2, 16
_NW = _NC * _NS
_RPW = _ROWS // _NW
_LANES = 16
_BLK = 4 * _LANES
_BLOCKS = _COLS // _BLK


def _mask_row(in_ref, out_ref):
    ia = lax.iota(jnp.int32, _LANES) * 4
    ib = ia + 1
    ic = ia + 2
    id_ = ia + 3
    one = jnp.float32(1.0)
    zero = jnp.float32(0.0)

    @plsc.parallel_loop(0, in_ref.shape[0] // _BLK, unroll=4)
    def block(blk):
        base = blk * _BLK
        in_blk = in_ref.at[pl.ds(base, _BLK)]
        out_blk = out_ref.at[pl.ds(base, _BLK)]
        a = plsc.load_gather(in_blk, [ia])
        b = plsc.load_gather(in_blk, [ib])
        c = plsc.load_gather(in_blk, [ic])
        d = plsc.load_gather(in_blk, [id_])
        x1 = a >= b
        x6 = c >= d
        h1 = jnp.where(x1, a, b)
        l1 = jnp.where(x1, b, a)
        h2 = jnp.where(x6, c, d)
        l2 = jnp.where(x6, d, c)
        y = h1 >= h2
        u = l1 >= h2
        v = h1 >= l2
        kh1 = y | v
        kl1 = y & u
        h1o = jnp.where(kh1, one, zero)
        l1o = jnp.where(kl1, one, zero)
        h2o = jnp.where(kl1, zero, one)
        l2o = jnp.where(kh1, zero, one)
        plsc.store_scatter(out_blk, [ia], jnp.where(x1, h1o, l1o))
        plsc.store_scatter(out_blk, [ib], jnp.where(x1, l1o, h1o))
        plsc.store_scatter(out_blk, [ic], jnp.where(x6, h2o, l2o))
        plsc.store_scatter(out_blk, [id_], jnp.where(x6, l2o, h2o))


_NSLOT = 4
_CH = _COLS // 2
_NJ = _RPW * 2 // _NSLOT


@functools.partial(
    pl.kernel,
    out_type=jax.ShapeDtypeStruct((_ROWS, _COLS), jnp.float32),
    mesh=plsc.VectorSubcoreMesh(core_axis_name="c", subcore_axis_name="s"),
    compiler_params=pltpu.CompilerParams(needs_layout_passes=False),
    scratch_types=(
        [pltpu.VMEM((_CH,), jnp.float32)] * (2 * _NSLOT)
        + [pltpu.SemaphoreType.DMA] * (2 * _NSLOT)
    ),
)
def _ampere_mask(in_hbm, out_hbm, *bufs_and_sems):
    ibufs = bufs_and_sems[:_NSLOT]
    obufs = bufs_and_sems[_NSLOT:2 * _NSLOT]
    isems = bufs_and_sems[2 * _NSLOT:3 * _NSLOT]
    osems = bufs_and_sems[3 * _NSLOT:]
    wid = lax.axis_index("s") * _NC + lax.axis_index("c")
    row0 = wid * _RPW

    def cref(hbm, j, s):
        return hbm.at[row0 + 2 * j + s // 2, pl.ds((s % 2) * _CH, _CH)]

    for s in range(_NSLOT):
        pltpu.async_copy(cref(in_hbm, 0, s), ibufs[s], isems[s])

    def step(j, carry):
        for s in range(_NSLOT):
            pltpu.make_async_copy(cref(in_hbm, j, s), ibufs[s], isems[s]).wait()

            @pl.when(j > 0)
            def _wait_prev_out():
                pltpu.make_async_copy(
                    obufs[s], cref(out_hbm, j - 1, s), osems[s]
                ).wait()

            _mask_row(ibufs[s], obufs[s])
            pltpu.async_copy(obufs[s], cref(out_hbm, j, s), osems[s])

            @pl.when(j + 1 < _NJ)
            def _prefetch_next_in():
                pltpu.async_copy(cref(in_hbm, j + 1, s), ibufs[s], isems[s])

        return carry

    lax.fori_loop(0, _NJ, step, 0)
    for s in range(_NSLOT):
        pltpu.make_async_copy(obufs[s], cref(out_hbm, _NJ - 1, s), osems[s]).wait()


def kernel(mask_scores, ampere_temperature):
    del ampere_temperature
    return _ampere_mask(mask_scores)
